# baseline (speedup 1.0000x reference)
import numpy as np
import jax
import jax.numpy as jnp
from jax.experimental import pallas as pl
from jax.experimental.pallas import tpu as pltpu

_B = 1024
_D = 128
_STEPS = 4
_CORES = 2
_BH = _B // _CORES
_M32 = 0xFFFFFFFF

_ROTS = ((13, 15, 26, 6), (17, 29, 16, 24))


def _tf2x32_np(k0, k1, x0, x1):
    k0, k1 = np.uint32(k0), np.uint32(k1)
    x0, x1 = x0.astype(np.uint32).copy(), x1.astype(np.uint32).copy()
    ks2 = np.uint32(k0 ^ k1 ^ np.uint32(0x1BD11BDA))
    inject = ((k1, ks2), (ks2, k0), (k0, k1), (k1, ks2), (ks2, k0))
    x0 += k0
    x1 += k1
    for i in range(5):
        for r in _ROTS[i % 2]:
            x0 = x0 + x1
            x1 = (x1 << np.uint32(r)) | (x1 >> np.uint32(32 - r))
            x1 = x1 ^ x0
        a, b = inject[i]
        x0 += a
        x1 += b + np.uint32(i + 1)
    return x0, x1


def _derive_step_keys():
    out = []
    for step in range(_STEPS):
        f0, f1 = _tf2x32_np(0, 42, np.zeros(1, np.uint32),
                            np.array([step], np.uint32))
        y0, y1 = _tf2x32_np(int(f0[0]), int(f1[0]), np.zeros(2, np.uint32),
                            np.arange(2, dtype=np.uint32))
        out.append(((int(y0[0]), int(y1[0])), (int(y0[1]), int(y1[1]))))
    return out


_STEP_KEYS = _derive_step_keys()


def _threefry_bits(k0, k1, counts):
    ks2 = (k0 ^ k1 ^ 0x1BD11BDA) & _M32
    ks = (k0, k1, ks2)
    inject = ((1, 2), (2, 0), (0, 1), (1, 2), (2, 0))
    x0 = jnp.full(counts.shape, np.uint32(k0), dtype=jnp.uint32)
    x1 = counts + np.uint32(k1)
    for i in range(5):
        for r in _ROTS[i % 2]:
            x0 = x0 + x1
            x1 = (x1 << np.uint32(r)) | (x1 >> np.uint32(32 - r))
            x1 = x1 ^ x0
        a, b = inject[i]
        x0 = x0 + np.uint32(ks[a])
        x1 = x1 + np.uint32((ks[b] + i + 1) & _M32)
    return x0 ^ x1


def _bits_to_u01(bits):
    f = jax.lax.bitcast_convert_type(
        (bits >> np.uint32(9)) | np.uint32(0x3F800000), jnp.float32)
    return f - 1.0


def _erfinv(u):
    w = -jnp.log((1.0 - u) * (1.0 + u))
    ws = w - 2.5
    p = jnp.float32(2.81022636e-08)
    for c in (3.43273939e-07, -3.5233877e-06, -4.39150654e-06, 0.00021858087,
              -0.00125372503, -0.00417768164, 0.246640727, 1.50140941):
        p = jnp.float32(c) + p * ws
    wl = jnp.sqrt(w) - 3.0
    q = jnp.float32(-0.000200214257)
    for c in (0.000100950558, 0.00134934322, -0.00367342844, 0.00573950773,
              -0.0076224613, 0.00943887047, 1.00167406, 2.83297682):
        q = jnp.float32(c) + q * wl
    return jnp.where(w < 5.0, p, q) * u


_LO = float(np.nextafter(np.float32(-1), np.float32(0)))
_SQRT2 = float(np.sqrt(np.float32(2.0)))


def _mh_kernel(x_hbm, mu_hbm, sigma_hbm, out_hbm,
               x_vm, mu_vm, sigma_vm, stage_vm, sem):
    pltpu.make_async_copy(sigma_hbm, sigma_vm, sem.at[0]).start()
    pltpu.make_async_copy(x_hbm, x_vm, sem.at[1]).start()
    pltpu.make_async_copy(mu_hbm, mu_vm, sem.at[2]).start()
    pltpu.make_async_copy(x_hbm, out_hbm.at[0], sem.at[3]).start()

    bh = _B
    counts = (jax.lax.broadcasted_iota(jnp.int32, (bh, _D), 0).astype(jnp.uint32)
              << np.uint32(7)) + \
        jax.lax.broadcasted_iota(jnp.int32, (bh, _D), 1).astype(jnp.uint32)

    noises, logus = [], []
    for s in range(_STEPS):
        (nk0, nk1), (uk0, uk1) = _STEP_KEYS[s]
        un = _bits_to_u01(_threefry_bits(nk0, nk1, counts))
        un = jnp.maximum(jnp.float32(_LO),
                         un * jnp.float32(1.0 - _LO) + jnp.float32(_LO))
        noises.append(jnp.float32(_SQRT2) * _erfinv(un))
        uu = jnp.maximum(jnp.float32(0.0),
                         _bits_to_u01(_threefry_bits(uk0, uk1, counts)))
        logus.append(jnp.log(uu))

    pltpu.make_async_copy(sigma_hbm, sigma_vm, sem.at[0]).wait()
    S = sigma_vm[...]
    eye = (jax.lax.broadcasted_iota(jnp.int32, (_D, _D), 0) ==
           jax.lax.broadcasted_iota(jnp.int32, (_D, _D), 1)).astype(jnp.float32)
    G = jnp.max(jnp.sum(jnp.abs(S), axis=1))
    c = 2.0 / (G + 0.1)
    Sb = S.astype(jnp.bfloat16)
    Pb = (c * eye).astype(jnp.bfloat16)
    for _ in range(9):
        T = jnp.dot(Sb, Pb, preferred_element_type=jnp.float32)
        Pb = (2.0 * Pb.astype(jnp.float32) -
              jnp.dot(Pb, T.astype(jnp.bfloat16),
                      preferred_element_type=jnp.float32)).astype(jnp.bfloat16)
    P = Pb.astype(jnp.float32)
    for _ in range(2):
        T = jnp.dot(S, P, preferred_element_type=jnp.float32,
                    precision=jax.lax.Precision.HIGHEST)
        P = 2.0 * P - jnp.dot(P, T, preferred_element_type=jnp.float32,
                              precision=jax.lax.Precision.HIGHEST)

    pjj = jnp.sum(P * eye, axis=0, keepdims=True)
    std = jnp.sqrt(1.0 / pjj)
    pltpu.make_async_copy(x_hbm, x_vm, sem.at[1]).wait()
    pltpu.make_async_copy(mu_hbm, mu_vm, sem.at[2]).wait()
    x0 = x_vm[...]
    mu = mu_vm[...]
    g = jnp.dot(x0 - mu, P, preferred_element_type=jnp.float32,
                precision=jax.lax.Precision.HIGHEST)

    x_cur = x0
    for s in range(_STEPS):
        x_til = x_cur + std * noises[s]
        dt = x_til - x0
        do = x_cur - x0
        lpa = -(dt - do) * (g + 0.5 * (dt + do) * pjj)
        x_cur = jnp.where(logus[s] < lpa, x_til, x_cur)
        stage_vm[s] = x_cur
        pltpu.make_async_copy(stage_vm.at[s], out_hbm.at[s + 1],
                              sem.at[4 + s]).start()
    pltpu.make_async_copy(x_hbm, out_hbm.at[0], sem.at[3]).wait()
    for s in range(_STEPS):
        pltpu.make_async_copy(stage_vm.at[s], out_hbm.at[s + 1],
                              sem.at[4 + s]).wait()


def kernel(X, X_mu, Sigma):
    return pl.pallas_call(
        _mh_kernel,
        in_specs=[pl.BlockSpec(memory_space=pl.ANY)] * 3,
        out_specs=pl.BlockSpec(memory_space=pl.ANY),
        out_shape=jax.ShapeDtypeStruct((_STEPS + 1, _B, _D), jnp.float32),
        scratch_shapes=[
            pltpu.VMEM((_B, _D), jnp.float32),
            pltpu.VMEM((1, _D), jnp.float32),
            pltpu.VMEM((_D, _D), jnp.float32),
            pltpu.VMEM((_STEPS, _B, _D), jnp.float32),
            pltpu.SemaphoreType.DMA((4 + _STEPS,)),
        ],
    )(X, X_mu[None, :], Sigma)


# speedup vs baseline: 1.7144x; 1.7144x over previous
import numpy as np
import jax
import jax.numpy as jnp
from jax.experimental import pallas as pl
from jax.experimental.pallas import tpu as pltpu

_B = 1024
_D = 128
_STEPS = 4
_CORES = 2
_BH = _B // _CORES
_M32 = 0xFFFFFFFF

_ROTS = ((13, 15, 26, 6), (17, 29, 16, 24))


def _tf2x32_np(k0, k1, x0, x1):
    k0, k1 = np.uint32(k0), np.uint32(k1)
    x0, x1 = x0.astype(np.uint32).copy(), x1.astype(np.uint32).copy()
    ks2 = np.uint32(k0 ^ k1 ^ np.uint32(0x1BD11BDA))
    inject = ((k1, ks2), (ks2, k0), (k0, k1), (k1, ks2), (ks2, k0))
    x0 += k0
    x1 += k1
    for i in range(5):
        for r in _ROTS[i % 2]:
            x0 = x0 + x1
            x1 = (x1 << np.uint32(r)) | (x1 >> np.uint32(32 - r))
            x1 = x1 ^ x0
        a, b = inject[i]
        x0 += a
        x1 += b + np.uint32(i + 1)
    return x0, x1


def _derive_step_keys():
    out = []
    for step in range(_STEPS):
        f0, f1 = _tf2x32_np(0, 42, np.zeros(1, np.uint32),
                            np.array([step], np.uint32))
        y0, y1 = _tf2x32_np(int(f0[0]), int(f1[0]), np.zeros(2, np.uint32),
                            np.arange(2, dtype=np.uint32))
        out.append(((int(y0[0]), int(y1[0])), (int(y0[1]), int(y1[1]))))
    return out


_STEP_KEYS = _derive_step_keys()


def _threefry_bits(k0, k1, counts):
    ks2 = (k0 ^ k1 ^ 0x1BD11BDA) & _M32
    ks = (k0, k1, ks2)
    inject = ((1, 2), (2, 0), (0, 1), (1, 2), (2, 0))
    x0 = jnp.full(counts.shape, np.uint32(k0), dtype=jnp.uint32)
    x1 = counts + np.uint32(k1)
    for i in range(5):
        for r in _ROTS[i % 2]:
            x0 = x0 + x1
            x1 = (x1 << np.uint32(r)) | (x1 >> np.uint32(32 - r))
            x1 = x1 ^ x0
        a, b = inject[i]
        x0 = x0 + np.uint32(ks[a])
        x1 = x1 + np.uint32((ks[b] + i + 1) & _M32)
    return x0 ^ x1


def _bits_to_u01(bits):
    f = jax.lax.bitcast_convert_type(
        (bits >> np.uint32(9)) | np.uint32(0x3F800000), jnp.float32)
    return f - 1.0


def _erfinv(u):
    w = -jnp.log((1.0 - u) * (1.0 + u))
    ws = w - 2.5
    p = jnp.float32(2.81022636e-08)
    for c in (3.43273939e-07, -3.5233877e-06, -4.39150654e-06, 0.00021858087,
              -0.00125372503, -0.00417768164, 0.246640727, 1.50140941):
        p = jnp.float32(c) + p * ws
    wl = jnp.sqrt(w) - 3.0
    q = jnp.float32(-0.000200214257)
    for c in (0.000100950558, 0.00134934322, -0.00367342844, 0.00573950773,
              -0.0076224613, 0.00943887047, 1.00167406, 2.83297682):
        q = jnp.float32(c) + q * wl
    return jnp.where(w < 5.0, p, q) * u


_LO = float(np.nextafter(np.float32(-1), np.float32(0)))
_SQRT2 = float(np.sqrt(np.float32(2.0)))


def _mh_kernel(x_ref, mu_ref, sigma_ref, out_ref):
    bh = _B
    counts = (jax.lax.broadcasted_iota(jnp.int32, (bh, _D), 0).astype(jnp.uint32)
              << np.uint32(7)) + \
        jax.lax.broadcasted_iota(jnp.int32, (bh, _D), 1).astype(jnp.uint32)

    noises, logus = [], []
    for s in range(_STEPS):
        (nk0, nk1), (uk0, uk1) = _STEP_KEYS[s]
        un = _bits_to_u01(_threefry_bits(nk0, nk1, counts))
        un = jnp.maximum(jnp.float32(_LO),
                         un * jnp.float32(1.0 - _LO) + jnp.float32(_LO))
        noises.append(jnp.float32(_SQRT2) * _erfinv(un))
        uu = jnp.maximum(jnp.float32(0.0),
                         _bits_to_u01(_threefry_bits(uk0, uk1, counts)))
        logus.append(jnp.log(uu))

    S = sigma_ref[...]
    eye = (jax.lax.broadcasted_iota(jnp.int32, (_D, _D), 0) ==
           jax.lax.broadcasted_iota(jnp.int32, (_D, _D), 1)).astype(jnp.float32)
    G = jnp.max(jnp.sum(jnp.abs(S), axis=1))
    c = 2.0 / (G + 0.1)
    Sb = S.astype(jnp.bfloat16)
    Pb = (c * eye).astype(jnp.bfloat16)
    for _ in range(9):
        T = jnp.dot(Sb, Pb, preferred_element_type=jnp.float32)
        Pb = (2.0 * Pb.astype(jnp.float32) -
              jnp.dot(Pb, T.astype(jnp.bfloat16),
                      preferred_element_type=jnp.float32)).astype(jnp.bfloat16)
    P = Pb.astype(jnp.float32)
    for _ in range(2):
        T = jnp.dot(S, P, preferred_element_type=jnp.float32,
                    precision=jax.lax.Precision.HIGHEST)
        P = 2.0 * P - jnp.dot(P, T, preferred_element_type=jnp.float32,
                              precision=jax.lax.Precision.HIGHEST)

    pjj = jnp.sum(P * eye, axis=0, keepdims=True)
    std = jnp.sqrt(1.0 / pjj)
    x0 = x_ref[...]
    mu = mu_ref[...]
    g = jnp.dot(x0 - mu, P, preferred_element_type=jnp.float32,
                precision=jax.lax.Precision.HIGHEST)

    out_ref[0] = x0
    x_cur = x0
    for s in range(_STEPS):
        x_til = x_cur + std * noises[s]
        dt = x_til - x0
        do = x_cur - x0
        lpa = -(dt - do) * (g + 0.5 * (dt + do) * pjj)
        x_cur = jnp.where(logus[s] < lpa, x_til, x_cur)
        out_ref[s + 1] = x_cur


def kernel(X, X_mu, Sigma):
    return pl.pallas_call(
        _mh_kernel,
        out_shape=jax.ShapeDtypeStruct((_STEPS + 1, _B, _D), jnp.float32),
    )(X, X_mu[None, :], Sigma)
